# baseline (speedup 1.0000x reference)
import functools

import jax
import jax.numpy as jnp
from jax import lax
from jax.experimental import pallas as pl


def _rollrows(x, s):
    return jnp.concatenate([x[s:, :], x[:s, :]], axis=0)


def _rolllanes(x, s):
    return jnp.concatenate([x[:, s:], x[:, :s]], axis=1)


def _bitonic_cm_kernel(x_ref, o_ref, *, L, CH):
    N = 1 << L
    R = N // 128
    La = L - 7
    lch = CH.bit_length() - 1
    nchunks = R // CH

    def cp(c, _):
        cb = c * CH
        o_ref[0, pl.ds(cb, CH), :] = x_ref[0, pl.ds(cb, CH), :]
        return 0

    lax.fori_loop(0, nchunks, cp, 0)

    row_iota = lax.broadcasted_iota(jnp.int32, (CH, 1), 0)
    lane_iota = lax.broadcasted_iota(jnp.int32, (1, 128), 1)

    def lanemask(bit):
        return ((lane_iota >> bit) & 1) == 0

    def rowmask(bit):
        return ((row_iota >> bit) & 1) == 0

    def stage_in_regs(x, cb, k, j):
        if j >= La:
            s = 1 << (j - La)
            lob = lanemask(j - La)
            part = jnp.where(lob, _rolllanes(x, s), _rolllanes(x, 128 - s))
            mn = jnp.minimum(x, part)
            mx = jnp.maximum(x, part)
            if k == L:
                tm = lob
            else:
                tm = lob == lanemask(k - La)
            return jnp.where(tm, mn, mx)
        s = 1 << j
        if k < lch:
            lob = rowmask(j)
            part = jnp.where(lob, _rollrows(x, s), _rollrows(x, CH - s))
            mn = jnp.minimum(x, part)
            mx = jnp.maximum(x, part)
            tm = lob == rowmask(k)
            return jnp.where(tm, mn, mx)
        g = CH // (2 * s)
        x4 = x.reshape(g, 2, s, 128)
        lo = x4[:, 0]
        hi = x4[:, 1]
        mn = jnp.minimum(lo, hi)
        mx = jnp.maximum(lo, hi)
        if k == L:
            nlo, nhi = mn, mx
        elif k >= La:
            am = lanemask(k - La)
            nlo = jnp.where(am, mn, mx)
            nhi = jnp.where(am, mx, mn)
        else:
            asc = ((cb >> k) & 1) == 0
            nlo = jnp.where(asc, mn, mx)
            nhi = jnp.where(asc, mx, mn)
        return jnp.concatenate(
            [nlo[:, None], nhi[:, None]], axis=1).reshape(CH, 128)

    def emit_chunk_run(k, js):
        if not js:
            return

        def body(c, _):
            cb = c * CH
            x = o_ref[0, pl.ds(cb, CH), :]
            for j in js:
                x = stage_in_regs(x, cb, k, j)
            o_ref[0, pl.ds(cb, CH), :] = x
            return 0

        lax.fori_loop(0, nchunks, body, 0)

    def emit_slab_stage(k, j):
        s = 1 << j
        ratio = s // CH
        for m in range((R // 2) // CH):
            g, t = divmod(m, ratio)
            lo_base = g * 2 * s + t * CH
            hi_base = lo_base + s
            lo = o_ref[0, pl.ds(lo_base, CH), :]
            hi = o_ref[0, pl.ds(hi_base, CH), :]
            mn = jnp.minimum(lo, hi)
            mx = jnp.maximum(lo, hi)
            if k == L:
                nlo, nhi = mn, mx
            elif k >= La:
                am = lanemask(k - La)
                nlo = jnp.where(am, mn, mx)
                nhi = jnp.where(am, mx, mn)
            else:
                if ((lo_base >> k) & 1) == 0:
                    nlo, nhi = mn, mx
                else:
                    nlo, nhi = mx, mn
            o_ref[0, pl.ds(lo_base, CH), :] = nlo
            o_ref[0, pl.ds(hi_base, CH), :] = nhi

    for k in range(1, L + 1):
        lane_js = [j for j in range(k - 1, -1, -1) if j >= La]
        slab_js = [j for j in range(min(k - 1, La - 1), -1, -1) if j >= lch]
        chunk_js = [j for j in range(min(k - 1, lch - 1), -1, -1)]
        emit_chunk_run(k, lane_js)
        for j in slab_js:
            emit_slab_stage(k, j)
        emit_chunk_run(k, chunk_js)


def _sort_padded_cm(x3, L, CH, interpret=False):
    B, R, _ = x3.shape
    return pl.pallas_call(
        functools.partial(_bitonic_cm_kernel, L=L, CH=CH),
        grid=(B,),
        in_specs=[pl.BlockSpec((1, R, 128), lambda i: (i, 0, 0))],
        out_specs=pl.BlockSpec((1, R, 128), lambda i: (i, 0, 0)),
        out_shape=jax.ShapeDtypeStruct((B, R, 128), jnp.float32),
        interpret=interpret,
    )(x3)


def kernel(x):
    B, n = x.shape
    L = max(8, (n - 1).bit_length())
    N = 1 << L
    R = N // 128
    CH = min(1024, R)
    xp = jnp.pad(x, ((0, 0), (0, N - n)), constant_values=jnp.float32(jnp.inf))
    out = _sort_padded_cm(xp.reshape(B, R, 128), L, CH)
    return out.transpose(0, 2, 1).reshape(B, N)[:, :n]


# speedup vs baseline: 1.0704x; 1.0704x over previous
import functools

import jax
import jax.numpy as jnp
from jax import lax
from jax.experimental import pallas as pl


def _rollrows(x, s):
    return jnp.concatenate([x[s:, :], x[:s, :]], axis=0)


def _rolllanes(x, s):
    return jnp.concatenate([x[:, s:], x[:, :s]], axis=1)


def _bitonic_cm_kernel(x_ref, o_ref, *, L, CH):
    N = 1 << L
    R = N // 128
    La = L - 7
    lch = CH.bit_length() - 1
    nchunks = R // CH

    def cp(c, _):
        cb = c * CH
        o_ref[0, pl.ds(cb, CH), :] = x_ref[0, pl.ds(cb, CH), :]
        return 0

    lax.fori_loop(0, nchunks, cp, 0)

    row_iota = lax.broadcasted_iota(jnp.int32, (CH, 1), 0)
    lane_iota = lax.broadcasted_iota(jnp.int32, (1, 128), 1)

    def lanemask(bit):
        return ((lane_iota >> bit) & 1) == 0

    def rowmask(bit):
        return ((row_iota >> bit) & 1) == 0

    def stage_in_regs(x, cb, k, j):
        if j >= La:
            s = 1 << (j - La)
            lob = lanemask(j - La)
            part = jnp.where(lob, _rolllanes(x, s), _rolllanes(x, 128 - s))
            mn = jnp.minimum(x, part)
            mx = jnp.maximum(x, part)
            if k == L:
                tm = lob
            else:
                tm = lob == lanemask(k - La)
            return jnp.where(tm, mn, mx)
        s = 1 << j
        if k < lch:
            lob = rowmask(j)
            part = jnp.where(lob, _rollrows(x, s), _rollrows(x, CH - s))
            mn = jnp.minimum(x, part)
            mx = jnp.maximum(x, part)
            tm = lob == rowmask(k)
            return jnp.where(tm, mn, mx)
        g = CH // (2 * s)
        x4 = x.reshape(g, 2, s, 128)
        lo = x4[:, 0]
        hi = x4[:, 1]
        mn = jnp.minimum(lo, hi)
        mx = jnp.maximum(lo, hi)
        if k == L:
            nlo, nhi = mn, mx
        elif k >= La:
            am = lanemask(k - La)
            nlo = jnp.where(am, mn, mx)
            nhi = jnp.where(am, mx, mn)
        else:
            asc = ((cb >> k) & 1) == 0
            nlo = jnp.where(asc, mn, mx)
            nhi = jnp.where(asc, mx, mn)
        return jnp.concatenate(
            [nlo[:, None], nhi[:, None]], axis=1).reshape(CH, 128)

    def emit_chunk_run(k, js):
        if not js:
            return

        def body(c, _):
            cb = c * CH
            x = o_ref[0, pl.ds(cb, CH), :]
            for j in js:
                x = stage_in_regs(x, cb, k, j)
            o_ref[0, pl.ds(cb, CH), :] = x
            return 0

        lax.fori_loop(0, nchunks, body, 0)

    def emit_slab_stage(k, j):
        s = 1 << j
        ratio = s // CH
        for m in range((R // 2) // CH):
            g, t = divmod(m, ratio)
            lo_base = g * 2 * s + t * CH
            hi_base = lo_base + s
            lo = o_ref[0, pl.ds(lo_base, CH), :]
            hi = o_ref[0, pl.ds(hi_base, CH), :]
            mn = jnp.minimum(lo, hi)
            mx = jnp.maximum(lo, hi)
            if k == L:
                nlo, nhi = mn, mx
            elif k >= La:
                am = lanemask(k - La)
                nlo = jnp.where(am, mn, mx)
                nhi = jnp.where(am, mx, mn)
            else:
                if ((lo_base >> k) & 1) == 0:
                    nlo, nhi = mn, mx
                else:
                    nlo, nhi = mx, mn
            o_ref[0, pl.ds(lo_base, CH), :] = nlo
            o_ref[0, pl.ds(hi_base, CH), :] = nhi

    for k in range(1, L + 1):
        lane_js = [j for j in range(k - 1, -1, -1) if j >= La]
        slab_js = [j for j in range(min(k - 1, La - 1), -1, -1) if j >= lch]
        chunk_js = [j for j in range(min(k - 1, lch - 1), -1, -1)]
        emit_chunk_run(k, lane_js)
        for j in slab_js:
            emit_slab_stage(k, j)
        emit_chunk_run(k, chunk_js)


def _sort_padded_cm(x3, L, CH, interpret=False):
    B, R, _ = x3.shape
    return pl.pallas_call(
        functools.partial(_bitonic_cm_kernel, L=L, CH=CH),
        grid=(B,),
        in_specs=[pl.BlockSpec((1, R, 128), lambda i: (i, 0, 0))],
        out_specs=pl.BlockSpec((1, R, 128), lambda i: (i, 0, 0)),
        out_shape=jax.ShapeDtypeStruct((B, R, 128), jnp.float32),
        interpret=interpret,
    )(x3)


def kernel(x):
    B, n = x.shape
    L = max(8, (n - 1).bit_length())
    N = 1 << L
    R = N // 128
    CH = min(512, R)
    xp = jnp.pad(x, ((0, 0), (0, N - n)), constant_values=jnp.float32(jnp.inf))
    out = _sort_padded_cm(xp.reshape(B, R, 128), L, CH)
    return out.transpose(0, 2, 1).reshape(B, N)[:, :n]


# speedup vs baseline: 1.1060x; 1.0333x over previous
import functools

import jax
import jax.numpy as jnp
from jax import lax
from jax.experimental import pallas as pl


def _rollrows(x, s):
    return jnp.concatenate([x[s:, :], x[:s, :]], axis=0)


def _rolllanes(x, s):
    return jnp.concatenate([x[:, s:], x[:, :s]], axis=1)


def _bitonic_cm_kernel(x_ref, o_ref, *, L, CH):
    N = 1 << L
    R = N // 128
    La = L - 7
    lch = CH.bit_length() - 1
    nchunks = R // CH

    def cp(c, _):
        cb = c * CH
        o_ref[0, pl.ds(cb, CH), :] = x_ref[0, pl.ds(cb, CH), :]
        return 0

    lax.fori_loop(0, nchunks, cp, 0)

    row_iota = lax.broadcasted_iota(jnp.int32, (CH, 1), 0)
    lane_iota = lax.broadcasted_iota(jnp.int32, (1, 128), 1)

    def lanemask(bit):
        return ((lane_iota >> bit) & 1) == 0

    def rowmask(bit):
        return ((row_iota >> bit) & 1) == 0

    def stage_in_regs(x, cb, k, j):
        if j >= La:
            s = 1 << (j - La)
            lob = lanemask(j - La)
            part = jnp.where(lob, _rolllanes(x, s), _rolllanes(x, 128 - s))
            mn = jnp.minimum(x, part)
            mx = jnp.maximum(x, part)
            if k == L:
                tm = lob
            else:
                tm = lob == lanemask(k - La)
            return jnp.where(tm, mn, mx)
        s = 1 << j
        g = CH // (2 * s)
        x4 = x.reshape(g, 2, s, 128)
        lo = x4[:, 0]
        hi = x4[:, 1]
        mn = jnp.minimum(lo, hi)
        mx = jnp.maximum(lo, hi)
        if k == L:
            nlo, nhi = mn, mx
        elif k >= La:
            am = lanemask(k - La)
            nlo = jnp.where(am, mn, mx)
            nhi = jnp.where(am, mx, mn)
        elif k < lch:
            gio = lax.broadcasted_iota(jnp.int32, (g, 1, 1), 0)
            am = ((gio >> (k - j - 1)) & 1) == 0
            nlo = jnp.where(am, mn, mx)
            nhi = jnp.where(am, mx, mn)
        else:
            asc = ((cb >> k) & 1) == 0
            nlo = jnp.where(asc, mn, mx)
            nhi = jnp.where(asc, mx, mn)
        return jnp.concatenate(
            [nlo[:, None], nhi[:, None]], axis=1).reshape(CH, 128)

    def emit_chunk_run(k, js):
        if not js:
            return

        def body(c, _):
            cb = c * CH
            x = o_ref[0, pl.ds(cb, CH), :]
            for j in js:
                x = stage_in_regs(x, cb, k, j)
            o_ref[0, pl.ds(cb, CH), :] = x
            return 0

        lax.fori_loop(0, nchunks, body, 0)

    def emit_slab_stage(k, j):
        s = 1 << j
        ratio = s // CH
        for m in range((R // 2) // CH):
            g, t = divmod(m, ratio)
            lo_base = g * 2 * s + t * CH
            hi_base = lo_base + s
            lo = o_ref[0, pl.ds(lo_base, CH), :]
            hi = o_ref[0, pl.ds(hi_base, CH), :]
            mn = jnp.minimum(lo, hi)
            mx = jnp.maximum(lo, hi)
            if k == L:
                nlo, nhi = mn, mx
            elif k >= La:
                am = lanemask(k - La)
                nlo = jnp.where(am, mn, mx)
                nhi = jnp.where(am, mx, mn)
            else:
                if ((lo_base >> k) & 1) == 0:
                    nlo, nhi = mn, mx
                else:
                    nlo, nhi = mx, mn
            o_ref[0, pl.ds(lo_base, CH), :] = nlo
            o_ref[0, pl.ds(hi_base, CH), :] = nhi

    for k in range(1, L + 1):
        lane_js = [j for j in range(k - 1, -1, -1) if j >= La]
        slab_js = [j for j in range(min(k - 1, La - 1), -1, -1) if j >= lch]
        chunk_js = [j for j in range(min(k - 1, lch - 1), -1, -1)]
        emit_chunk_run(k, lane_js)
        for j in slab_js:
            emit_slab_stage(k, j)
        emit_chunk_run(k, chunk_js)


def _sort_padded_cm(x3, L, CH, interpret=False):
    B, R, _ = x3.shape
    return pl.pallas_call(
        functools.partial(_bitonic_cm_kernel, L=L, CH=CH),
        grid=(B,),
        in_specs=[pl.BlockSpec((1, R, 128), lambda i: (i, 0, 0))],
        out_specs=pl.BlockSpec((1, R, 128), lambda i: (i, 0, 0)),
        out_shape=jax.ShapeDtypeStruct((B, R, 128), jnp.float32),
        interpret=interpret,
    )(x3)


def kernel(x):
    B, n = x.shape
    L = max(8, (n - 1).bit_length())
    N = 1 << L
    R = N // 128
    CH = min(512, R)
    xp = jnp.pad(x, ((0, 0), (0, N - n)), constant_values=jnp.float32(jnp.inf))
    out = _sort_padded_cm(xp.reshape(B, R, 128), L, CH)
    return out.transpose(0, 2, 1).reshape(B, N)[:, :n]
